# baseline (speedup 1.0000x reference)
import jax
import jax.numpy as jnp
from jax.experimental import pallas as pl

_SMPL_PARENTS = (-1, 0, 0, 0, 1, 2, 3, 4, 5, 6, 7, 8, 9, 9, 9, 12, 13,
                 14, 16, 17, 18, 19, 20, 21)
_J = 24
_HID = 96
_HEADS = 3
_OUT_CH = _HID // _HEADS
_JH = _J * _HEADS
_F = 512


def _encoder_block(src_ref, pre_w_ref, pre_b_ref, gat_w_ref, gat_b_ref,
                   ba_ref, exp_ref, out_ref):
    pre_w = pre_w_ref[...]
    pre_b = pre_b_ref[...]
    gat_w = gat_w_ref[...]
    gat_b = gat_b_ref[...]
    ba = ba_ref[...]
    expand = exp_ref[...]

    xh = []
    for j in range(_J):
        x = src_ref[j]
        x = jnp.maximum(
            jnp.dot(x, pre_w, preferred_element_type=jnp.float32) + pre_b, 0.0)
        xh.append(jnp.dot(x, gat_w, preferred_element_type=jnp.float32))

    xh_all = jnp.concatenate(xh, axis=1)
    a3 = jnp.dot(xh_all, ba, preferred_element_type=jnp.float32)
    a_s = a3[:, 0:_JH]
    a_d = a3[:, _JH:2 * _JH]
    a_sp = a3[:, 2 * _JH:3 * _JH]

    al_s = a_s + a_d
    al_p = a_sp + a_d
    al_s = jnp.where(al_s > 0, al_s, 0.2 * al_s)
    al_p = jnp.where(al_p > 0, al_p, 0.2 * al_p)
    m = jnp.maximum(al_s, al_p)
    es = jnp.exp(al_s - m)
    ep = jnp.exp(al_p - m)
    inv = 1.0 / (es + ep + 1e-16)
    csn = es * inv
    cpn = ep * inv

    outs = []
    for j in range(_J):
        p = _SMPL_PARENTS[j]
        if p < 0:
            o = xh[j]
        else:
            cs = jnp.dot(csn[:, j * _HEADS:(j + 1) * _HEADS], expand,
                         preferred_element_type=jnp.float32)
            cp = jnp.dot(cpn[:, j * _HEADS:(j + 1) * _HEADS], expand,
                         preferred_element_type=jnp.float32)
            o = cs * xh[j] + cp * xh[p]
        outs.append(jnp.maximum(o + gat_b, 0.0))
    out_ref[...] = jnp.concatenate(outs, axis=1)


def kernel(src, pre_w, pre_b, gat_w, att_src, att_dst, gat_b):
    N, L, D = src.shape
    NL = N * L
    src_t = src.reshape(NL, _J, 3).transpose(1, 0, 2)
    eye3 = jnp.eye(_HEADS, dtype=jnp.float32)
    base_s = (att_src[:, :, None] * eye3[:, None, :]).reshape(_HID, _HEADS)
    base_d = (att_dst[:, :, None] * eye3[:, None, :]).reshape(_HID, _HEADS)
    eye_j = jnp.eye(_J, dtype=jnp.float32)
    parents = jnp.array([max(p, 0) for p in _SMPL_PARENTS], dtype=jnp.int32)
    par_sel = jnp.zeros((_J, _J), jnp.float32).at[
        parents, jnp.arange(_J)].set(1.0)
    ba = jnp.concatenate([
        jnp.kron(eye_j, base_s),
        jnp.kron(eye_j, base_d),
        jnp.kron(par_sel, base_s),
    ], axis=1)
    expand = jnp.repeat(eye3, _OUT_CH, axis=1)

    out = pl.pallas_call(
        _encoder_block,
        grid=(NL // _F,),
        in_specs=[
            pl.BlockSpec((_J, _F, 3), lambda i: (0, i, 0)),
            pl.BlockSpec((3, _HID), lambda i: (0, 0)),
            pl.BlockSpec((1, _HID), lambda i: (0, 0)),
            pl.BlockSpec((_HID, _HID), lambda i: (0, 0)),
            pl.BlockSpec((1, _HID), lambda i: (0, 0)),
            pl.BlockSpec((_J * _HID, 3 * _JH), lambda i: (0, 0)),
            pl.BlockSpec((_HEADS, _HID), lambda i: (0, 0)),
        ],
        out_specs=pl.BlockSpec((_F, _J * _HID), lambda i: (i, 0)),
        out_shape=jax.ShapeDtypeStruct((NL, _J * _HID), jnp.float32),
    )(src_t, pre_w, pre_b.reshape(1, _HID), gat_w,
      gat_b.reshape(1, _HID), ba, expand)
    return out.reshape(N, L, _J * _HID)


# speedup vs baseline: 1.2486x; 1.2486x over previous
import jax
import jax.numpy as jnp
from jax.experimental import pallas as pl

_SMPL_PARENTS = (-1, 0, 0, 0, 1, 2, 3, 4, 5, 6, 7, 8, 9, 9, 9, 12, 13,
                 14, 16, 17, 18, 19, 20, 21)
_J = 24
_HID = 96
_HEADS = 3
_OUT_CH = _HID // _HEADS
_JH = _J * _HEADS
_F = 512


def _encoder_block(src_ref, pre_w_ref, pre_b_ref, gat_w_ref, gat_b_ref,
                   ba_ref, exp_ref, out_ref):
    pre_w = pre_w_ref[...]
    pre_b = pre_b_ref[...]
    gat_w = gat_w_ref[...]
    gat_b = gat_b_ref[...]
    ba = ba_ref[...]
    expand = exp_ref[...]

    x_all = jnp.maximum(
        jnp.dot(src_ref[...], pre_w, preferred_element_type=jnp.float32)
        + pre_b, 0.0)
    xh = [jnp.dot(x_all[:, j * _HID:(j + 1) * _HID], gat_w,
                  preferred_element_type=jnp.float32) for j in range(_J)]

    xh_all = jnp.concatenate(xh, axis=1)
    a3 = jnp.dot(xh_all, ba, preferred_element_type=jnp.float32)
    a_s = a3[:, 0:_JH]
    a_d = a3[:, _JH:2 * _JH]
    a_sp = a3[:, 2 * _JH:3 * _JH]

    al_s = a_s + a_d
    al_p = a_sp + a_d
    al_s = jnp.where(al_s > 0, al_s, 0.2 * al_s)
    al_p = jnp.where(al_p > 0, al_p, 0.2 * al_p)
    m = jnp.maximum(al_s, al_p)
    es = jnp.exp(al_s - m)
    ep = jnp.exp(al_p - m)
    inv = 1.0 / (es + ep + 1e-16)
    csn = es * inv
    cpn = ep * inv

    outs = []
    for j in range(_J):
        p = _SMPL_PARENTS[j]
        if p < 0:
            o = xh[j]
        else:
            cs = jnp.dot(csn[:, j * _HEADS:(j + 1) * _HEADS], expand,
                         preferred_element_type=jnp.float32)
            cp = jnp.dot(cpn[:, j * _HEADS:(j + 1) * _HEADS], expand,
                         preferred_element_type=jnp.float32)
            o = cs * xh[j] + cp * xh[p]
        outs.append(jnp.maximum(o + gat_b, 0.0))
    out_ref[...] = jnp.concatenate(outs, axis=1)


def kernel(src, pre_w, pre_b, gat_w, att_src, att_dst, gat_b):
    N, L, D = src.shape
    NL = N * L
    src2 = src.reshape(NL, _J * 3)
    eye3 = jnp.eye(_HEADS, dtype=jnp.float32)
    eye_j = jnp.eye(_J, dtype=jnp.float32)
    pre_w_big = jnp.kron(eye_j, pre_w)
    pre_b_big = jnp.tile(pre_b, (_J,)).reshape(1, _J * _HID)
    base_s = (att_src[:, :, None] * eye3[:, None, :]).reshape(_HID, _HEADS)
    base_d = (att_dst[:, :, None] * eye3[:, None, :]).reshape(_HID, _HEADS)
    parents = jnp.array([max(p, 0) for p in _SMPL_PARENTS], dtype=jnp.int32)
    par_sel = jnp.zeros((_J, _J), jnp.float32).at[
        parents, jnp.arange(_J)].set(1.0)
    ba = jnp.concatenate([
        jnp.kron(eye_j, base_s),
        jnp.kron(eye_j, base_d),
        jnp.kron(par_sel, base_s),
    ], axis=1)
    expand = jnp.repeat(eye3, _OUT_CH, axis=1)

    out = pl.pallas_call(
        _encoder_block,
        grid=(NL // _F,),
        in_specs=[
            pl.BlockSpec((_F, _J * 3), lambda i: (i, 0)),
            pl.BlockSpec((_J * 3, _J * _HID), lambda i: (0, 0)),
            pl.BlockSpec((1, _J * _HID), lambda i: (0, 0)),
            pl.BlockSpec((_HID, _HID), lambda i: (0, 0)),
            pl.BlockSpec((1, _HID), lambda i: (0, 0)),
            pl.BlockSpec((_J * _HID, 3 * _JH), lambda i: (0, 0)),
            pl.BlockSpec((_HEADS, _HID), lambda i: (0, 0)),
        ],
        out_specs=pl.BlockSpec((_F, _J * _HID), lambda i: (i, 0)),
        out_shape=jax.ShapeDtypeStruct((NL, _J * _HID), jnp.float32),
    )(src2, pre_w_big, pre_b_big, gat_w,
      gat_b.reshape(1, _HID), ba, expand)
    return out.reshape(N, L, _J * _HID)
